# baseline (speedup 1.0000x reference)
import functools

import jax
import jax.numpy as jnp
from jax import lax
from jax.experimental import pallas as pl
from jax.experimental.pallas import tpu as pltpu
from jax.experimental.pallas import tpu_sc as plsc

N = 10000
E = 320000
CAT = 1000

NW = 32
NSUB = 16
NP = 10240
ECH = 128
NCHE = 80
EPW = NCHE * ECH
GCH = 64
NCHG = 5
ROWS_PER_SUB = NP // NSUB

CH0 = 128
CH1 = 32


def _sc_mesh():
    return plsc.VectorSubcoreMesh(core_axis_name="c", subcore_axis_name="s")


def _sc_front(idx_u, idx_c, dst_pad, emb_user, emb_cat, zeros16, ones16):
    @functools.partial(
        pl.kernel,
        mesh=_sc_mesh(),
        compiler_params=pltpu.CompilerParams(use_tc_tiling_on_sc=False),
        out_type=[
            jax.ShapeDtypeStruct((NP, 64), jnp.float32),
            jax.ShapeDtypeStruct((NP, 32), jnp.float32),
            jax.ShapeDtypeStruct((2, NP, 16), jnp.float32),
        ],
        scratch_types=[
            pltpu.VMEM((GCH,), jnp.int32),
            pltpu.VMEM((GCH, 64), jnp.float32),
            pltpu.VMEM((GCH, 32), jnp.float32),
            pltpu.VMEM((ECH,), jnp.int32),
            pltpu.VMEM((ECH,), jnp.int32),
            pltpu.VMEM((ECH, 16), jnp.float32),
            pltpu.VMEM_SHARED((NP, 16), jnp.float32),
            pltpu.SemaphoreType.DMA,
            pltpu.SemaphoreType.DMA,
            pltpu.SemaphoreType.DMA,
        ],
    )
    def k(idx_u_hbm, idx_c_hbm, dst_hbm, emb_u_hbm, emb_c_hbm,
          zeros16_hbm, ones16_hbm,
          eu_hbm, ec_hbm, deg_hbm,
          idxg, bufu, bufc, idxe0, idxe1, ones_v, deg_sh, sem, sd0, sd1):
        cid = lax.axis_index("c")
        sid = lax.axis_index("s")
        wid = cid * NSUB + sid
        gbase = wid * (NCHG * GCH)
        ebase = wid * EPW
        idxe = (idxe0, idxe1)
        sd = (sd0, sd1)

        def start_didx(j, b):
            pltpu.async_copy(dst_hbm.at[pl.ds(ebase + j * ECH, ECH)],
                             idxe[b], sd[b])

        def wait_didx(b):
            pltpu.make_async_copy(dst_hbm.at[pl.ds(ebase, ECH)],
                                  idxe[b], sd[b]).wait()

        start_didx(0, 0)
        start_didx(1, 1)
        pltpu.sync_copy(zeros16_hbm.at[pl.ds(sid * ROWS_PER_SUB, ROWS_PER_SUB)],
                        deg_sh.at[pl.ds(sid * ROWS_PER_SUB, ROWS_PER_SUB)])
        pltpu.sync_copy(ones16_hbm, ones_v)
        plsc.subcore_barrier()

        def deg_body(i, carry):
            for b in range(2):
                j = 2 * i + b
                wait_didx(b)
                pltpu.sync_copy(ones_v, deg_sh.at[idxe[b]], add=True)

                @pl.when(j + 2 < NCHE)
                def _():
                    start_didx(j + 2, b)
            return carry

        lax.fori_loop(0, NCHE // 2, deg_body, 0)

        def g_body(j, carry):
            base = gbase + j * GCH
            pltpu.sync_copy(idx_u_hbm.at[pl.ds(base, GCH)], idxg)
            pltpu.async_copy(emb_u_hbm.at[idxg], bufu, sem).wait()
            pltpu.sync_copy(bufu, eu_hbm.at[pl.ds(base, GCH)])
            pltpu.sync_copy(idx_c_hbm.at[pl.ds(base, GCH)], idxg)
            pltpu.async_copy(emb_c_hbm.at[idxg], bufc, sem).wait()
            pltpu.sync_copy(bufc, ec_hbm.at[pl.ds(base, GCH)])
            return carry

        lax.fori_loop(0, NCHG, g_body, 0)

        plsc.subcore_barrier()
        pltpu.sync_copy(deg_sh.at[pl.ds(sid * ROWS_PER_SUB, ROWS_PER_SUB)],
                        deg_hbm.at[cid, pl.ds(sid * ROWS_PER_SUB, ROWS_PER_SUB)])

    return k(idx_u, idx_c, dst_pad, emb_user, emb_cat, zeros16, ones16)


def _sc_spmm_single(y, src_pad, dst_pad, zeros, D):
    nche = 2 * NCHE

    @functools.partial(
        pl.kernel,
        mesh=_sc_mesh(),
        compiler_params=pltpu.CompilerParams(use_tc_tiling_on_sc=False),
        out_type=jax.ShapeDtypeStruct((NP, D), jnp.float32),
        scratch_types=[
            pltpu.VMEM((ECH,), jnp.int32),
            pltpu.VMEM((ECH,), jnp.int32),
            pltpu.VMEM((ECH,), jnp.int32),
            pltpu.VMEM((ECH,), jnp.int32),
            pltpu.VMEM((ECH, D), jnp.float32),
            pltpu.VMEM((ECH, D), jnp.float32),
            pltpu.VMEM_SHARED((NP, D), jnp.float32),
            pltpu.SemaphoreType.DMA,
            pltpu.SemaphoreType.DMA,
            pltpu.SemaphoreType.DMA,
            pltpu.SemaphoreType.DMA,
            pltpu.SemaphoreType.DMA,
            pltpu.SemaphoreType.DMA,
        ],
    )
    def k(y_hbm, src_hbm, dst_hbm, zeros_hbm, z_hbm,
          sidx0, sidx1, didx0, didx1, buf0, buf1, z_sh,
          ss0, ss1, sd0, sd1, sg0, sg1):
        cid = lax.axis_index("c")
        sid = lax.axis_index("s")
        ebase = sid * (nche * ECH)
        sidx = (sidx0, sidx1)
        didx = (didx0, didx1)
        buf = (buf0, buf1)
        ss = (ss0, ss1)
        sd = (sd0, sd1)
        sg = (sg0, sg1)

        def start_idx(j, b):
            pltpu.async_copy(src_hbm.at[pl.ds(ebase + j * ECH, ECH)],
                             sidx[b], ss[b])
            pltpu.async_copy(dst_hbm.at[pl.ds(ebase + j * ECH, ECH)],
                             didx[b], sd[b])

        def wait_sidx(b):
            pltpu.make_async_copy(src_hbm.at[pl.ds(ebase, ECH)],
                                  sidx[b], ss[b]).wait()

        def wait_didx(b):
            pltpu.make_async_copy(dst_hbm.at[pl.ds(ebase, ECH)],
                                  didx[b], sd[b]).wait()

        @pl.when(cid == 0)
        def _():
            start_idx(0, 0)
            start_idx(1, 1)
            pltpu.sync_copy(
                zeros_hbm.at[pl.ds(sid * ROWS_PER_SUB, ROWS_PER_SUB)],
                z_sh.at[pl.ds(sid * ROWS_PER_SUB, ROWS_PER_SUB)])

        plsc.subcore_barrier()

        @pl.when(cid == 0)
        def _():
            wait_sidx(0)
            pltpu.async_copy(y_hbm.at[sidx[0]], buf[0], sg[0])

            def body(i, carry):
                for b in range(2):
                    j = 2 * i + b
                    nb = 1 - b
                    pltpu.make_async_copy(y_hbm.at[sidx[b]], buf[b],
                                          sg[b]).wait()

                    @pl.when(j + 1 < nche)
                    def _():
                        wait_sidx(nb)
                        pltpu.async_copy(y_hbm.at[sidx[nb]], buf[nb], sg[nb])

                    wait_didx(b)
                    pltpu.sync_copy(buf[b], z_sh.at[didx[b]], add=True)

                    @pl.when(j + 2 < nche)
                    def _():
                        start_idx(j + 2, b)
                return carry

            lax.fori_loop(0, nche // 2, body, 0)

        plsc.subcore_barrier()

        @pl.when(cid == 0)
        def _():
            pltpu.sync_copy(z_sh.at[pl.ds(sid * ROWS_PER_SUB, ROWS_PER_SUB)],
                            z_hbm.at[pl.ds(sid * ROWS_PER_SUB, ROWS_PER_SUB)])

    return k(y, src_pad, dst_pad, zeros)


def _sc_spmm(y, src_pad, dst_pad, zeros, D):
    @functools.partial(
        pl.kernel,
        mesh=_sc_mesh(),
        compiler_params=pltpu.CompilerParams(use_tc_tiling_on_sc=False),
        out_type=jax.ShapeDtypeStruct((2, NP, D), jnp.float32),
        scratch_types=[
            pltpu.VMEM((ECH,), jnp.int32),
            pltpu.VMEM((ECH,), jnp.int32),
            pltpu.VMEM((ECH,), jnp.int32),
            pltpu.VMEM((ECH,), jnp.int32),
            pltpu.VMEM((ECH, D), jnp.float32),
            pltpu.VMEM((ECH, D), jnp.float32),
            pltpu.VMEM_SHARED((NP, D), jnp.float32),
            pltpu.SemaphoreType.DMA,
            pltpu.SemaphoreType.DMA,
            pltpu.SemaphoreType.DMA,
            pltpu.SemaphoreType.DMA,
            pltpu.SemaphoreType.DMA,
            pltpu.SemaphoreType.DMA,
        ],
    )
    def k(y_hbm, src_hbm, dst_hbm, zeros_hbm, z_hbm,
          sidx0, sidx1, didx0, didx1, buf0, buf1, z_sh,
          ss0, ss1, sd0, sd1, sg0, sg1):
        cid = lax.axis_index("c")
        sid = lax.axis_index("s")
        nche = jnp.where(cid == 0, CH0, CH1)
        ebase = jnp.where(cid == 0, sid * CH0, NSUB * CH0 + sid * CH1) * ECH
        sidx = (sidx0, sidx1)
        didx = (didx0, didx1)
        buf = (buf0, buf1)
        ss = (ss0, ss1)
        sd = (sd0, sd1)
        sg = (sg0, sg1)

        def start_idx(j, b):
            pltpu.async_copy(src_hbm.at[pl.ds(ebase + j * ECH, ECH)],
                             sidx[b], ss[b])
            pltpu.async_copy(dst_hbm.at[pl.ds(ebase + j * ECH, ECH)],
                             didx[b], sd[b])

        def wait_sidx(b):
            pltpu.make_async_copy(src_hbm.at[pl.ds(ebase, ECH)],
                                  sidx[b], ss[b]).wait()

        def wait_didx(b):
            pltpu.make_async_copy(dst_hbm.at[pl.ds(ebase, ECH)],
                                  didx[b], sd[b]).wait()

        start_idx(0, 0)
        start_idx(1, 1)
        pltpu.sync_copy(zeros_hbm.at[pl.ds(sid * ROWS_PER_SUB, ROWS_PER_SUB)],
                        z_sh.at[pl.ds(sid * ROWS_PER_SUB, ROWS_PER_SUB)])
        plsc.subcore_barrier()
        wait_sidx(0)
        pltpu.async_copy(y_hbm.at[sidx[0]], buf[0], sg[0])

        def body(i, carry):
            for b in range(2):
                j = 2 * i + b
                nb = 1 - b
                pltpu.make_async_copy(y_hbm.at[sidx[b]], buf[b], sg[b]).wait()

                @pl.when(j + 1 < nche)
                def _():
                    wait_sidx(nb)
                    pltpu.async_copy(y_hbm.at[sidx[nb]], buf[nb], sg[nb])

                wait_didx(b)
                pltpu.sync_copy(buf[b], z_sh.at[didx[b]], add=True)

                @pl.when(j + 2 < nche)
                def _():
                    start_idx(j + 2, b)
            return carry

        lax.fori_loop(0, nche // 2, body, 0)

        plsc.subcore_barrier()
        pltpu.sync_copy(z_sh.at[pl.ds(sid * ROWS_PER_SUB, ROWS_PER_SUB)],
                        z_hbm.at[cid, pl.ds(sid * ROWS_PER_SUB, ROWS_PER_SUB)])

    return k(y, src_pad, dst_pad, zeros)


_RB = 1280
_NRB = NP // _RB


def _tc_front(eu, ec, kn, fl, lm, degA, degB, emb_known, W_user, b_user,
              emb_mask, W_mask, b_mask, W_cat, b_cat, W0):
    def body(eu_r, ec_r, kn_r, fl_r, lm_r, dA_r, dB_r, ek_r, Wu_r, bu_r,
             em_r, Wm_r, bm_r, Wc_r, bc_r, W0_r, y0_r):
        eu_b = eu_r[...]
        kn_b = kn_r[...]
        ksel = jnp.where(kn_b == 0, ek_r[0:1, :], ek_r[1:2, :])
        uf = jnp.maximum(eu_b + ksel, 0.0) @ Wu_r[...] + bu_r[...]
        cf = jnp.maximum(ec_r[...], 0.0) @ Wc_r[...] + bc_r[...]
        mrows = jax.nn.sigmoid(jnp.maximum(em_r[...], 0.0) @ Wm_r[...] + bm_r[...])
        mf = jnp.where(lm_r[...] == 0, mrows[0:1, :], mrows[1:2, :])
        x = jnp.where(fl_r[...] == 0, uf, cf) * mf
        deg = dA_r[...][:, 0:1] + dB_r[...][:, 0:1] + 1.0
        dinv = lax.rsqrt(deg)
        y0_r[...] = dinv * (x @ W0_r[...])

    full = lambda shape: pl.BlockSpec(shape, lambda i: (0, 0))
    return pl.pallas_call(
        body,
        grid=(_NRB,),
        in_specs=[
            pl.BlockSpec((_RB, 64), lambda i: (i, 0)),
            pl.BlockSpec((_RB, 32), lambda i: (i, 0)),
            pl.BlockSpec((_RB, 1), lambda i: (i, 0)),
            pl.BlockSpec((_RB, 1), lambda i: (i, 0)),
            pl.BlockSpec((_RB, 1), lambda i: (i, 0)),
            pl.BlockSpec((_RB, 16), lambda i: (i, 0)),
            pl.BlockSpec((_RB, 16), lambda i: (i, 0)),
            full((2, 64)),
            full((64, 128)),
            full((1, 128)),
            full((2, 64)),
            full((64, 128)),
            full((1, 128)),
            full((32, 128)),
            full((1, 128)),
            full((128, 128)),
        ],
        out_specs=pl.BlockSpec((_RB, 128), lambda i: (i, 0)),
        out_shape=jax.ShapeDtypeStruct((NP, 128), jnp.float32),
    )(eu, ec, kn, fl, lm, degA, degB, emb_known, W_user, b_user,
      emb_mask, W_mask, b_mask, W_cat, b_cat, W0)


def _tc_mid(z0a, y0, degA, degB, b0, W2):
    def body(za_r, y0_r, dA_r, dB_r, b0_r, W2_r, y2_r):
        deg = dA_r[...][:, 0:1] + dB_r[...][:, 0:1] + 1.0
        dinv = lax.rsqrt(deg)
        out0 = dinv * (za_r[...] + y0_r[...]) + b0_r[...]
        x1 = jnp.maximum(out0, 0.0)
        y2_r[...] = dinv * (x1 @ W2_r[...])

    full = lambda shape: pl.BlockSpec(shape, lambda i: (0, 0))
    return pl.pallas_call(
        body,
        grid=(_NRB,),
        in_specs=[
            pl.BlockSpec((_RB, 128), lambda i: (i, 0)),
            pl.BlockSpec((_RB, 128), lambda i: (i, 0)),
            pl.BlockSpec((_RB, 16), lambda i: (i, 0)),
            pl.BlockSpec((_RB, 16), lambda i: (i, 0)),
            full((1, 128)),
            full((128, 64)),
        ],
        out_specs=pl.BlockSpec((_RB, 64), lambda i: (i, 0)),
        out_shape=jax.ShapeDtypeStruct((NP, 64), jnp.float32),
    )(z0a, y0, degA, degB, b0, W2)


def _tc_tail(z2a, z2b, y2, degA, degB, b2):
    def body(za_r, zb_r, y2_r, dA_r, dB_r, b2_r, out_r):
        deg = dA_r[...][:, 0:1] + dB_r[...][:, 0:1] + 1.0
        dinv = lax.rsqrt(deg)
        out_r[...] = dinv * (za_r[...] + zb_r[...] + y2_r[...]) + b2_r[...]

    full = lambda shape: pl.BlockSpec(shape, lambda i: (0, 0))
    return pl.pallas_call(
        body,
        grid=(_NRB,),
        in_specs=[
            pl.BlockSpec((_RB, 64), lambda i: (i, 0)),
            pl.BlockSpec((_RB, 64), lambda i: (i, 0)),
            pl.BlockSpec((_RB, 64), lambda i: (i, 0)),
            pl.BlockSpec((_RB, 16), lambda i: (i, 0)),
            pl.BlockSpec((_RB, 16), lambda i: (i, 0)),
            full((1, 64)),
        ],
        out_specs=pl.BlockSpec((_RB, 64), lambda i: (i, 0)),
        out_shape=jax.ShapeDtypeStruct((NP, 64), jnp.float32),
    )(z2a, z2b, y2, degA, degB, b2)


def kernel(edges, features, label_masks, emb_user, emb_known, W_user, b_user,
           emb_mask, W_mask, b_mask, emb_cat, W_cat, b_cat,
           W0, b0, W1, b1, W2, b2):
    idx = features[:, 0]
    known = features[:, 1]
    flag = features[:, 2]

    pad_n = NP - N
    idx_u = jnp.concatenate([idx, jnp.zeros((pad_n,), jnp.int32)])
    idx_c = jnp.concatenate([jnp.clip(idx, 0, CAT - 1),
                             jnp.zeros((pad_n,), jnp.int32)])
    kn = jnp.concatenate([known, jnp.zeros((pad_n,), jnp.int32)]).reshape(NP, 1)
    fl = jnp.concatenate([flag, jnp.zeros((pad_n,), jnp.int32)]).reshape(NP, 1)
    lm = jnp.concatenate([label_masks,
                          jnp.zeros((pad_n,), jnp.int32)]).reshape(NP, 1)

    pad_e = NW * EPW - E
    pad_idx = jnp.full((pad_e,), N, jnp.int32)
    src_pad = jnp.concatenate([edges[0], pad_idx])
    dst_pad = jnp.concatenate([edges[1], pad_idx])

    zeros128 = jnp.zeros((NP, 128), jnp.float32)
    zeros64 = jnp.zeros((NP, 64), jnp.float32)
    zeros16 = jnp.zeros((NP, 16), jnp.float32)
    ones16 = jnp.ones((ECH, 16), jnp.float32)

    eu, ec, degp = _sc_front(idx_u, idx_c, dst_pad, emb_user, emb_cat,
                             zeros16, ones16)
    degA, degB = degp[0], degp[1]

    y0 = _tc_front(eu, ec, kn, fl, lm, degA, degB, emb_known, W_user,
                   b_user.reshape(1, -1), emb_mask, W_mask,
                   b_mask.reshape(1, -1), W_cat, b_cat.reshape(1, -1), W0)

    z0 = _sc_spmm_single(y0, src_pad, dst_pad, zeros128, 128)
    y2 = _tc_mid(z0, y0, degA, degB, b0.reshape(1, -1), W2)
    z2 = _sc_spmm(y2, src_pad, dst_pad, zeros64, 64)
    out = _tc_tail(z2[0], z2[1], y2, degA, degB, b2.reshape(1, -1))
    return out[:N]


# speedup vs baseline: 1.1985x; 1.1985x over previous
import functools

import jax
import jax.numpy as jnp
from jax import lax
from jax.experimental import pallas as pl
from jax.experimental.pallas import tpu as pltpu
from jax.experimental.pallas import tpu_sc as plsc

N = 10000
E = 320000
CAT = 1000

NW = 32
NSUB = 16
NP = 10240
ECH = 128
NCHE = 80
EPW = NCHE * ECH
GCH = 64
NCHG = 5
ROWS_PER_SUB = NP // NSUB

CH0 = 144
CH1 = 16


def _sc_mesh():
    return plsc.VectorSubcoreMesh(core_axis_name="c", subcore_axis_name="s")


def _sc_front(idx_u, idx_c, dst_pad, emb_user, emb_cat, zeros16, ones16):
    @functools.partial(
        pl.kernel,
        mesh=_sc_mesh(),
        compiler_params=pltpu.CompilerParams(use_tc_tiling_on_sc=False),
        out_type=[
            jax.ShapeDtypeStruct((NP, 64), jnp.float32),
            jax.ShapeDtypeStruct((NP, 32), jnp.float32),
            jax.ShapeDtypeStruct((2, NP, 16), jnp.float32),
        ],
        scratch_types=[
            pltpu.VMEM((GCH,), jnp.int32),
            pltpu.VMEM((GCH, 64), jnp.float32),
            pltpu.VMEM((GCH, 32), jnp.float32),
            pltpu.VMEM((ECH,), jnp.int32),
            pltpu.VMEM((ECH,), jnp.int32),
            pltpu.VMEM((ECH, 16), jnp.float32),
            pltpu.VMEM_SHARED((NP, 16), jnp.float32),
            pltpu.SemaphoreType.DMA,
            pltpu.SemaphoreType.DMA,
            pltpu.SemaphoreType.DMA,
        ],
    )
    def k(idx_u_hbm, idx_c_hbm, dst_hbm, emb_u_hbm, emb_c_hbm,
          zeros16_hbm, ones16_hbm,
          eu_hbm, ec_hbm, deg_hbm,
          idxg, bufu, bufc, idxe0, idxe1, ones_v, deg_sh, sem, sd0, sd1):
        cid = lax.axis_index("c")
        sid = lax.axis_index("s")
        wid = cid * NSUB + sid
        gbase = wid * (NCHG * GCH)
        ebase = wid * EPW
        idxe = (idxe0, idxe1)
        sd = (sd0, sd1)

        def start_didx(j, b):
            pltpu.async_copy(dst_hbm.at[pl.ds(ebase + j * ECH, ECH)],
                             idxe[b], sd[b])

        def wait_didx(b):
            pltpu.make_async_copy(dst_hbm.at[pl.ds(ebase, ECH)],
                                  idxe[b], sd[b]).wait()

        start_didx(0, 0)
        start_didx(1, 1)
        pltpu.sync_copy(zeros16_hbm.at[pl.ds(sid * ROWS_PER_SUB, ROWS_PER_SUB)],
                        deg_sh.at[pl.ds(sid * ROWS_PER_SUB, ROWS_PER_SUB)])
        pltpu.sync_copy(ones16_hbm, ones_v)
        plsc.subcore_barrier()

        def deg_body(i, carry):
            for b in range(2):
                j = 2 * i + b
                wait_didx(b)
                pltpu.sync_copy(ones_v, deg_sh.at[idxe[b]], add=True)

                @pl.when(j + 2 < NCHE)
                def _():
                    start_didx(j + 2, b)
            return carry

        lax.fori_loop(0, NCHE // 2, deg_body, 0)

        def g_body(j, carry):
            base = gbase + j * GCH
            pltpu.sync_copy(idx_u_hbm.at[pl.ds(base, GCH)], idxg)
            pltpu.async_copy(emb_u_hbm.at[idxg], bufu, sem).wait()
            pltpu.sync_copy(bufu, eu_hbm.at[pl.ds(base, GCH)])
            pltpu.sync_copy(idx_c_hbm.at[pl.ds(base, GCH)], idxg)
            pltpu.async_copy(emb_c_hbm.at[idxg], bufc, sem).wait()
            pltpu.sync_copy(bufc, ec_hbm.at[pl.ds(base, GCH)])
            return carry

        lax.fori_loop(0, NCHG, g_body, 0)

        plsc.subcore_barrier()
        pltpu.sync_copy(deg_sh.at[pl.ds(sid * ROWS_PER_SUB, ROWS_PER_SUB)],
                        deg_hbm.at[cid, pl.ds(sid * ROWS_PER_SUB, ROWS_PER_SUB)])

    return k(idx_u, idx_c, dst_pad, emb_user, emb_cat, zeros16, ones16)


def _sc_spmm(y, src_pad, dst_pad, zeros, D):
    @functools.partial(
        pl.kernel,
        mesh=_sc_mesh(),
        compiler_params=pltpu.CompilerParams(use_tc_tiling_on_sc=False),
        out_type=jax.ShapeDtypeStruct((2, NP, D), jnp.float32),
        scratch_types=[
            pltpu.VMEM((ECH,), jnp.int32),
            pltpu.VMEM((ECH,), jnp.int32),
            pltpu.VMEM((ECH,), jnp.int32),
            pltpu.VMEM((ECH,), jnp.int32),
            pltpu.VMEM((ECH, D), jnp.float32),
            pltpu.VMEM((ECH, D), jnp.float32),
            pltpu.VMEM_SHARED((NP, D), jnp.float32),
            pltpu.SemaphoreType.DMA,
            pltpu.SemaphoreType.DMA,
            pltpu.SemaphoreType.DMA,
            pltpu.SemaphoreType.DMA,
            pltpu.SemaphoreType.DMA,
            pltpu.SemaphoreType.DMA,
        ],
    )
    def k(y_hbm, src_hbm, dst_hbm, zeros_hbm, z_hbm,
          sidx0, sidx1, didx0, didx1, buf0, buf1, z_sh,
          ss0, ss1, sd0, sd1, sg0, sg1):
        cid = lax.axis_index("c")
        sid = lax.axis_index("s")
        nche = jnp.where(cid == 0, CH0, CH1)
        ebase = jnp.where(cid == 0, sid * CH0, NSUB * CH0 + sid * CH1) * ECH
        sidx = (sidx0, sidx1)
        didx = (didx0, didx1)
        buf = (buf0, buf1)
        ss = (ss0, ss1)
        sd = (sd0, sd1)
        sg = (sg0, sg1)

        def start_idx(j, b):
            pltpu.async_copy(src_hbm.at[pl.ds(ebase + j * ECH, ECH)],
                             sidx[b], ss[b])
            pltpu.async_copy(dst_hbm.at[pl.ds(ebase + j * ECH, ECH)],
                             didx[b], sd[b])

        def wait_sidx(b):
            pltpu.make_async_copy(src_hbm.at[pl.ds(ebase, ECH)],
                                  sidx[b], ss[b]).wait()

        def wait_didx(b):
            pltpu.make_async_copy(dst_hbm.at[pl.ds(ebase, ECH)],
                                  didx[b], sd[b]).wait()

        start_idx(0, 0)
        start_idx(1, 1)
        pltpu.sync_copy(zeros_hbm.at[pl.ds(sid * ROWS_PER_SUB, ROWS_PER_SUB)],
                        z_sh.at[pl.ds(sid * ROWS_PER_SUB, ROWS_PER_SUB)])
        plsc.subcore_barrier()
        wait_sidx(0)
        pltpu.async_copy(y_hbm.at[sidx[0]], buf[0], sg[0])

        def body(i, carry):
            for b in range(2):
                j = 2 * i + b
                nb = 1 - b
                pltpu.make_async_copy(y_hbm.at[sidx[b]], buf[b], sg[b]).wait()

                @pl.when(j + 1 < nche)
                def _():
                    wait_sidx(nb)
                    pltpu.async_copy(y_hbm.at[sidx[nb]], buf[nb], sg[nb])

                wait_didx(b)
                pltpu.sync_copy(buf[b], z_sh.at[didx[b]], add=True)

                @pl.when(j + 2 < nche)
                def _():
                    start_idx(j + 2, b)
            return carry

        lax.fori_loop(0, nche // 2, body, 0)

        plsc.subcore_barrier()
        pltpu.sync_copy(z_sh.at[pl.ds(sid * ROWS_PER_SUB, ROWS_PER_SUB)],
                        z_hbm.at[cid, pl.ds(sid * ROWS_PER_SUB, ROWS_PER_SUB)])

    return k(y, src_pad, dst_pad, zeros)


_RB = 1280
_NRB = NP // _RB


def _tc_front(eu, ec, kn, fl, lm, degA, degB, emb_known, W_user, b_user,
              emb_mask, W_mask, b_mask, W_cat, b_cat, W0):
    def body(eu_r, ec_r, kn_r, fl_r, lm_r, dA_r, dB_r, ek_r, Wu_r, bu_r,
             em_r, Wm_r, bm_r, Wc_r, bc_r, W0_r, y0_r):
        eu_b = eu_r[...]
        kn_b = kn_r[...]
        ksel = jnp.where(kn_b == 0, ek_r[0:1, :], ek_r[1:2, :])
        uf = jnp.maximum(eu_b + ksel, 0.0) @ Wu_r[...] + bu_r[...]
        cf = jnp.maximum(ec_r[...], 0.0) @ Wc_r[...] + bc_r[...]
        mrows = jax.nn.sigmoid(jnp.maximum(em_r[...], 0.0) @ Wm_r[...] + bm_r[...])
        mf = jnp.where(lm_r[...] == 0, mrows[0:1, :], mrows[1:2, :])
        x = jnp.where(fl_r[...] == 0, uf, cf) * mf
        deg = dA_r[...][:, 0:1] + dB_r[...][:, 0:1] + 1.0
        dinv = lax.rsqrt(deg)
        y0_r[...] = dinv * (x @ W0_r[...])

    full = lambda shape: pl.BlockSpec(shape, lambda i: (0, 0))
    return pl.pallas_call(
        body,
        grid=(_NRB,),
        in_specs=[
            pl.BlockSpec((_RB, 64), lambda i: (i, 0)),
            pl.BlockSpec((_RB, 32), lambda i: (i, 0)),
            pl.BlockSpec((_RB, 1), lambda i: (i, 0)),
            pl.BlockSpec((_RB, 1), lambda i: (i, 0)),
            pl.BlockSpec((_RB, 1), lambda i: (i, 0)),
            pl.BlockSpec((_RB, 16), lambda i: (i, 0)),
            pl.BlockSpec((_RB, 16), lambda i: (i, 0)),
            full((2, 64)),
            full((64, 128)),
            full((1, 128)),
            full((2, 64)),
            full((64, 128)),
            full((1, 128)),
            full((32, 128)),
            full((1, 128)),
            full((128, 128)),
        ],
        out_specs=pl.BlockSpec((_RB, 128), lambda i: (i, 0)),
        out_shape=jax.ShapeDtypeStruct((NP, 128), jnp.float32),
    )(eu, ec, kn, fl, lm, degA, degB, emb_known, W_user, b_user,
      emb_mask, W_mask, b_mask, W_cat, b_cat, W0)


def _tc_mid(z0a, z0b, y0, degA, degB, b0, W2):
    def body(za_r, zb_r, y0_r, dA_r, dB_r, b0_r, W2_r, y2_r):
        deg = dA_r[...][:, 0:1] + dB_r[...][:, 0:1] + 1.0
        dinv = lax.rsqrt(deg)
        out0 = dinv * (za_r[...] + zb_r[...] + y0_r[...]) + b0_r[...]
        x1 = jnp.maximum(out0, 0.0)
        y2_r[...] = dinv * (x1 @ W2_r[...])

    full = lambda shape: pl.BlockSpec(shape, lambda i: (0, 0))
    return pl.pallas_call(
        body,
        grid=(_NRB,),
        in_specs=[
            pl.BlockSpec((_RB, 128), lambda i: (i, 0)),
            pl.BlockSpec((_RB, 128), lambda i: (i, 0)),
            pl.BlockSpec((_RB, 128), lambda i: (i, 0)),
            pl.BlockSpec((_RB, 16), lambda i: (i, 0)),
            pl.BlockSpec((_RB, 16), lambda i: (i, 0)),
            full((1, 128)),
            full((128, 64)),
        ],
        out_specs=pl.BlockSpec((_RB, 64), lambda i: (i, 0)),
        out_shape=jax.ShapeDtypeStruct((NP, 64), jnp.float32),
    )(z0a, z0b, y0, degA, degB, b0, W2)


def _tc_tail(z2a, z2b, y2, degA, degB, b2):
    def body(za_r, zb_r, y2_r, dA_r, dB_r, b2_r, out_r):
        deg = dA_r[...][:, 0:1] + dB_r[...][:, 0:1] + 1.0
        dinv = lax.rsqrt(deg)
        out_r[...] = dinv * (za_r[...] + zb_r[...] + y2_r[...]) + b2_r[...]

    full = lambda shape: pl.BlockSpec(shape, lambda i: (0, 0))
    return pl.pallas_call(
        body,
        grid=(_NRB,),
        in_specs=[
            pl.BlockSpec((_RB, 64), lambda i: (i, 0)),
            pl.BlockSpec((_RB, 64), lambda i: (i, 0)),
            pl.BlockSpec((_RB, 64), lambda i: (i, 0)),
            pl.BlockSpec((_RB, 16), lambda i: (i, 0)),
            pl.BlockSpec((_RB, 16), lambda i: (i, 0)),
            full((1, 64)),
        ],
        out_specs=pl.BlockSpec((_RB, 64), lambda i: (i, 0)),
        out_shape=jax.ShapeDtypeStruct((NP, 64), jnp.float32),
    )(z2a, z2b, y2, degA, degB, b2)


def kernel(edges, features, label_masks, emb_user, emb_known, W_user, b_user,
           emb_mask, W_mask, b_mask, emb_cat, W_cat, b_cat,
           W0, b0, W1, b1, W2, b2):
    idx = features[:, 0]
    known = features[:, 1]
    flag = features[:, 2]

    pad_n = NP - N
    idx_u = jnp.concatenate([idx, jnp.zeros((pad_n,), jnp.int32)])
    idx_c = jnp.concatenate([jnp.clip(idx, 0, CAT - 1),
                             jnp.zeros((pad_n,), jnp.int32)])
    kn = jnp.concatenate([known, jnp.zeros((pad_n,), jnp.int32)]).reshape(NP, 1)
    fl = jnp.concatenate([flag, jnp.zeros((pad_n,), jnp.int32)]).reshape(NP, 1)
    lm = jnp.concatenate([label_masks,
                          jnp.zeros((pad_n,), jnp.int32)]).reshape(NP, 1)

    pad_e = NW * EPW - E
    pad_idx = jnp.full((pad_e,), N, jnp.int32)
    src_pad = jnp.concatenate([edges[0], pad_idx])
    dst_pad = jnp.concatenate([edges[1], pad_idx])

    zeros128 = jnp.zeros((NP, 128), jnp.float32)
    zeros64 = jnp.zeros((NP, 64), jnp.float32)
    zeros16 = jnp.zeros((NP, 16), jnp.float32)
    ones16 = jnp.ones((ECH, 16), jnp.float32)

    eu, ec, degp = _sc_front(idx_u, idx_c, dst_pad, emb_user, emb_cat,
                             zeros16, ones16)
    degA, degB = degp[0], degp[1]

    y0 = _tc_front(eu, ec, kn, fl, lm, degA, degB, emb_known, W_user,
                   b_user.reshape(1, -1), emb_mask, W_mask,
                   b_mask.reshape(1, -1), W_cat, b_cat.reshape(1, -1), W0)

    z0 = _sc_spmm(y0, src_pad, dst_pad, zeros128, 128)
    y2 = _tc_mid(z0[0], z0[1], y0, degA, degB, b0.reshape(1, -1), W2)
    z2 = _sc_spmm(y2, src_pad, dst_pad, zeros64, 64)
    out = _tc_tail(z2[0], z2[1], y2, degA, degB, b2.reshape(1, -1))
    return out[:N]


# speedup vs baseline: 1.2138x; 1.0128x over previous
import functools

import jax
import jax.numpy as jnp
from jax import lax
from jax.experimental import pallas as pl
from jax.experimental.pallas import tpu as pltpu
from jax.experimental.pallas import tpu_sc as plsc

N = 10000
E = 320000
CAT = 1000

NW = 32
NSUB = 16
NP = 10240
ECH = 128
NCHE = 80
EPW = NCHE * ECH
GCH = 64
NCHG = 5
ROWS_PER_SUB = NP // NSUB

CH0 = 144
CH1 = 16


def _sc_mesh():
    return plsc.VectorSubcoreMesh(core_axis_name="c", subcore_axis_name="s")


def _sc_front(idx_u, idx_c, dst_pad, emb_user, emb_cat, zeros16, ones16):
    @functools.partial(
        pl.kernel,
        mesh=_sc_mesh(),
        compiler_params=pltpu.CompilerParams(use_tc_tiling_on_sc=False),
        out_type=[
            jax.ShapeDtypeStruct((NP, 64), jnp.float32),
            jax.ShapeDtypeStruct((NP, 32), jnp.float32),
            jax.ShapeDtypeStruct((2, NP, 16), jnp.float32),
        ],
        scratch_types=(
            [pltpu.VMEM((GCH,), jnp.int32),
             pltpu.VMEM((GCH, 64), jnp.float32),
             pltpu.VMEM((GCH, 32), jnp.float32)]
            + [pltpu.VMEM((ECH,), jnp.int32) for _ in range(8)]
            + [pltpu.VMEM((ECH, 16), jnp.float32),
               pltpu.VMEM_SHARED((NP, 16), jnp.float32)]
            + [pltpu.SemaphoreType.DMA for _ in range(1 + 8 + 4)]
        ),
    )
    def k(idx_u_hbm, idx_c_hbm, dst_hbm, emb_u_hbm, emb_c_hbm,
          zeros16_hbm, ones16_hbm,
          eu_hbm, ec_hbm, deg_hbm, *scr):
        idxg, bufu, bufc = scr[0], scr[1], scr[2]
        idxe = scr[3:11]
        ones_v = scr[11]
        deg_sh = scr[12]
        sem = scr[13]
        sd = scr[14:22]
        sc = scr[22:26]

        cid = lax.axis_index("c")
        sid = lax.axis_index("s")
        wid = cid * NSUB + sid
        gbase = wid * (NCHG * GCH)
        ebase = wid * EPW

        def start_didx(j, s):
            pltpu.async_copy(dst_hbm.at[pl.ds(ebase + j * ECH, ECH)],
                             idxe[s], sd[s])

        def wait_didx(s):
            pltpu.make_async_copy(dst_hbm.at[pl.ds(ebase, ECH)],
                                  idxe[s], sd[s]).wait()

        def wait_scat(s8, sc4):
            pltpu.make_async_copy(ones_v, deg_sh.at[idxe[s8]],
                                  sc[sc4]).wait()

        for t in range(4):
            start_didx(t, t)
        pltpu.sync_copy(zeros16_hbm.at[pl.ds(sid * ROWS_PER_SUB, ROWS_PER_SUB)],
                        deg_sh.at[pl.ds(sid * ROWS_PER_SUB, ROWS_PER_SUB)])
        pltpu.sync_copy(ones16_hbm, ones_v)
        plsc.subcore_barrier()

        def deg_body(i, carry):
            for kk in range(8):
                j = 8 * i + kk
                s = kk
                wait_didx(s)
                pltpu.async_copy(ones_v, deg_sh.at[idxe[s]], sc[kk % 4],
                                 add=True)
                if kk >= 2:
                    wait_scat((kk + 6) % 8, (kk + 2) % 4)
                else:
                    @pl.when(i > 0)
                    def _():
                        wait_scat((kk + 6) % 8, (kk + 2) % 4)

                @pl.when(j + 4 < NCHE)
                def _():
                    start_didx(j + 4, (kk + 4) % 8)
            return carry

        lax.fori_loop(0, NCHE // 8, deg_body, 0)
        wait_scat(6, 2)
        wait_scat(7, 3)

        def g_body(j, carry):
            base = gbase + j * GCH
            pltpu.sync_copy(idx_u_hbm.at[pl.ds(base, GCH)], idxg)
            pltpu.async_copy(emb_u_hbm.at[idxg], bufu, sem).wait()
            pltpu.sync_copy(bufu, eu_hbm.at[pl.ds(base, GCH)])
            pltpu.sync_copy(idx_c_hbm.at[pl.ds(base, GCH)], idxg)
            pltpu.async_copy(emb_c_hbm.at[idxg], bufc, sem).wait()
            pltpu.sync_copy(bufc, ec_hbm.at[pl.ds(base, GCH)])
            return carry

        lax.fori_loop(0, NCHG, g_body, 0)

        plsc.subcore_barrier()
        pltpu.sync_copy(deg_sh.at[pl.ds(sid * ROWS_PER_SUB, ROWS_PER_SUB)],
                        deg_hbm.at[cid, pl.ds(sid * ROWS_PER_SUB, ROWS_PER_SUB)])

    return k(idx_u, idx_c, dst_pad, emb_user, emb_cat, zeros16, ones16)


def _sc_spmm(y, src_pad, dst_pad, zeros, D):
    @functools.partial(
        pl.kernel,
        mesh=_sc_mesh(),
        compiler_params=pltpu.CompilerParams(use_tc_tiling_on_sc=False),
        out_type=jax.ShapeDtypeStruct((2, NP, D), jnp.float32),
        scratch_types=[
            pltpu.VMEM((ECH,), jnp.int32),
            pltpu.VMEM((ECH,), jnp.int32),
            pltpu.VMEM((ECH,), jnp.int32),
            pltpu.VMEM((ECH,), jnp.int32),
            pltpu.VMEM((ECH, D), jnp.float32),
            pltpu.VMEM((ECH, D), jnp.float32),
            pltpu.VMEM_SHARED((NP, D), jnp.float32),
            pltpu.SemaphoreType.DMA,
            pltpu.SemaphoreType.DMA,
            pltpu.SemaphoreType.DMA,
            pltpu.SemaphoreType.DMA,
            pltpu.SemaphoreType.DMA,
            pltpu.SemaphoreType.DMA,
        ],
    )
    def k(y_hbm, src_hbm, dst_hbm, zeros_hbm, z_hbm,
          sidx0, sidx1, didx0, didx1, buf0, buf1, z_sh,
          ss0, ss1, sd0, sd1, sg0, sg1):
        cid = lax.axis_index("c")
        sid = lax.axis_index("s")
        nche = jnp.where(cid == 0, CH0, CH1)
        ebase = jnp.where(cid == 0, sid * CH0, NSUB * CH0 + sid * CH1) * ECH
        sidx = (sidx0, sidx1)
        didx = (didx0, didx1)
        buf = (buf0, buf1)
        ss = (ss0, ss1)
        sd = (sd0, sd1)
        sg = (sg0, sg1)

        def start_idx(j, b):
            pltpu.async_copy(src_hbm.at[pl.ds(ebase + j * ECH, ECH)],
                             sidx[b], ss[b])
            pltpu.async_copy(dst_hbm.at[pl.ds(ebase + j * ECH, ECH)],
                             didx[b], sd[b])

        def wait_sidx(b):
            pltpu.make_async_copy(src_hbm.at[pl.ds(ebase, ECH)],
                                  sidx[b], ss[b]).wait()

        def wait_didx(b):
            pltpu.make_async_copy(dst_hbm.at[pl.ds(ebase, ECH)],
                                  didx[b], sd[b]).wait()

        start_idx(0, 0)
        start_idx(1, 1)
        pltpu.sync_copy(zeros_hbm.at[pl.ds(sid * ROWS_PER_SUB, ROWS_PER_SUB)],
                        z_sh.at[pl.ds(sid * ROWS_PER_SUB, ROWS_PER_SUB)])
        plsc.subcore_barrier()
        wait_sidx(0)
        pltpu.async_copy(y_hbm.at[sidx[0]], buf[0], sg[0])

        def body(i, carry):
            for b in range(2):
                j = 2 * i + b
                nb = 1 - b
                pltpu.make_async_copy(y_hbm.at[sidx[b]], buf[b], sg[b]).wait()

                @pl.when(j + 1 < nche)
                def _():
                    wait_sidx(nb)
                    pltpu.async_copy(y_hbm.at[sidx[nb]], buf[nb], sg[nb])

                wait_didx(b)
                pltpu.sync_copy(buf[b], z_sh.at[didx[b]], add=True)

                @pl.when(j + 2 < nche)
                def _():
                    start_idx(j + 2, b)
            return carry

        lax.fori_loop(0, nche // 2, body, 0)

        plsc.subcore_barrier()
        pltpu.sync_copy(z_sh.at[pl.ds(sid * ROWS_PER_SUB, ROWS_PER_SUB)],
                        z_hbm.at[cid, pl.ds(sid * ROWS_PER_SUB, ROWS_PER_SUB)])

    return k(y, src_pad, dst_pad, zeros)


_RB = 1280
_NRB = NP // _RB


def _tc_front(eu, ec, kn, fl, lm, degA, degB, emb_known, W_user, b_user,
              emb_mask, W_mask, b_mask, W_cat, b_cat, W0):
    def body(eu_r, ec_r, kn_r, fl_r, lm_r, dA_r, dB_r, ek_r, Wu_r, bu_r,
             em_r, Wm_r, bm_r, Wc_r, bc_r, W0_r, y0_r):
        eu_b = eu_r[...]
        kn_b = kn_r[...]
        ksel = jnp.where(kn_b == 0, ek_r[0:1, :], ek_r[1:2, :])
        uf = jnp.maximum(eu_b + ksel, 0.0) @ Wu_r[...] + bu_r[...]
        cf = jnp.maximum(ec_r[...], 0.0) @ Wc_r[...] + bc_r[...]
        mrows = jax.nn.sigmoid(jnp.maximum(em_r[...], 0.0) @ Wm_r[...] + bm_r[...])
        mf = jnp.where(lm_r[...] == 0, mrows[0:1, :], mrows[1:2, :])
        x = jnp.where(fl_r[...] == 0, uf, cf) * mf
        deg = dA_r[...][:, 0:1] + dB_r[...][:, 0:1] + 1.0
        dinv = lax.rsqrt(deg)
        y0_r[...] = dinv * (x @ W0_r[...])

    full = lambda shape: pl.BlockSpec(shape, lambda i: (0, 0))
    return pl.pallas_call(
        body,
        grid=(_NRB,),
        in_specs=[
            pl.BlockSpec((_RB, 64), lambda i: (i, 0)),
            pl.BlockSpec((_RB, 32), lambda i: (i, 0)),
            pl.BlockSpec((_RB, 1), lambda i: (i, 0)),
            pl.BlockSpec((_RB, 1), lambda i: (i, 0)),
            pl.BlockSpec((_RB, 1), lambda i: (i, 0)),
            pl.BlockSpec((_RB, 16), lambda i: (i, 0)),
            pl.BlockSpec((_RB, 16), lambda i: (i, 0)),
            full((2, 64)),
            full((64, 128)),
            full((1, 128)),
            full((2, 64)),
            full((64, 128)),
            full((1, 128)),
            full((32, 128)),
            full((1, 128)),
            full((128, 128)),
        ],
        out_specs=pl.BlockSpec((_RB, 128), lambda i: (i, 0)),
        out_shape=jax.ShapeDtypeStruct((NP, 128), jnp.float32),
    )(eu, ec, kn, fl, lm, degA, degB, emb_known, W_user, b_user,
      emb_mask, W_mask, b_mask, W_cat, b_cat, W0)


def _tc_mid(z0a, z0b, y0, degA, degB, b0, W2):
    def body(za_r, zb_r, y0_r, dA_r, dB_r, b0_r, W2_r, y2_r):
        deg = dA_r[...][:, 0:1] + dB_r[...][:, 0:1] + 1.0
        dinv = lax.rsqrt(deg)
        out0 = dinv * (za_r[...] + zb_r[...] + y0_r[...]) + b0_r[...]
        x1 = jnp.maximum(out0, 0.0)
        y2_r[...] = dinv * (x1 @ W2_r[...])

    full = lambda shape: pl.BlockSpec(shape, lambda i: (0, 0))
    return pl.pallas_call(
        body,
        grid=(_NRB,),
        in_specs=[
            pl.BlockSpec((_RB, 128), lambda i: (i, 0)),
            pl.BlockSpec((_RB, 128), lambda i: (i, 0)),
            pl.BlockSpec((_RB, 128), lambda i: (i, 0)),
            pl.BlockSpec((_RB, 16), lambda i: (i, 0)),
            pl.BlockSpec((_RB, 16), lambda i: (i, 0)),
            full((1, 128)),
            full((128, 64)),
        ],
        out_specs=pl.BlockSpec((_RB, 64), lambda i: (i, 0)),
        out_shape=jax.ShapeDtypeStruct((NP, 64), jnp.float32),
    )(z0a, z0b, y0, degA, degB, b0, W2)


def _tc_tail(z2a, z2b, y2, degA, degB, b2):
    def body(za_r, zb_r, y2_r, dA_r, dB_r, b2_r, out_r):
        deg = dA_r[...][:, 0:1] + dB_r[...][:, 0:1] + 1.0
        dinv = lax.rsqrt(deg)
        out_r[...] = dinv * (za_r[...] + zb_r[...] + y2_r[...]) + b2_r[...]

    full = lambda shape: pl.BlockSpec(shape, lambda i: (0, 0))
    return pl.pallas_call(
        body,
        grid=(_NRB,),
        in_specs=[
            pl.BlockSpec((_RB, 64), lambda i: (i, 0)),
            pl.BlockSpec((_RB, 64), lambda i: (i, 0)),
            pl.BlockSpec((_RB, 64), lambda i: (i, 0)),
            pl.BlockSpec((_RB, 16), lambda i: (i, 0)),
            pl.BlockSpec((_RB, 16), lambda i: (i, 0)),
            full((1, 64)),
        ],
        out_specs=pl.BlockSpec((_RB, 64), lambda i: (i, 0)),
        out_shape=jax.ShapeDtypeStruct((NP, 64), jnp.float32),
    )(z2a, z2b, y2, degA, degB, b2)


def kernel(edges, features, label_masks, emb_user, emb_known, W_user, b_user,
           emb_mask, W_mask, b_mask, emb_cat, W_cat, b_cat,
           W0, b0, W1, b1, W2, b2):
    idx = features[:, 0]
    known = features[:, 1]
    flag = features[:, 2]

    pad_n = NP - N
    idx_u = jnp.concatenate([idx, jnp.zeros((pad_n,), jnp.int32)])
    idx_c = jnp.concatenate([jnp.clip(idx, 0, CAT - 1),
                             jnp.zeros((pad_n,), jnp.int32)])
    kn = jnp.concatenate([known, jnp.zeros((pad_n,), jnp.int32)]).reshape(NP, 1)
    fl = jnp.concatenate([flag, jnp.zeros((pad_n,), jnp.int32)]).reshape(NP, 1)
    lm = jnp.concatenate([label_masks,
                          jnp.zeros((pad_n,), jnp.int32)]).reshape(NP, 1)

    pad_e = NW * EPW - E
    pad_idx = jnp.full((pad_e,), N, jnp.int32)
    src_pad = jnp.concatenate([edges[0], pad_idx])
    dst_pad = jnp.concatenate([edges[1], pad_idx])

    zeros128 = jnp.zeros((NP, 128), jnp.float32)
    zeros64 = jnp.zeros((NP, 64), jnp.float32)
    zeros16 = jnp.zeros((NP, 16), jnp.float32)
    ones16 = jnp.ones((ECH, 16), jnp.float32)

    eu, ec, degp = _sc_front(idx_u, idx_c, dst_pad, emb_user, emb_cat,
                             zeros16, ones16)
    degA, degB = degp[0], degp[1]

    y0 = _tc_front(eu, ec, kn, fl, lm, degA, degB, emb_known, W_user,
                   b_user.reshape(1, -1), emb_mask, W_mask,
                   b_mask.reshape(1, -1), W_cat, b_cat.reshape(1, -1), W0)

    z0 = _sc_spmm(y0, src_pad, dst_pad, zeros128, 128)
    y2 = _tc_mid(z0[0], z0[1], y0, degA, degB, b0.reshape(1, -1), W2)
    z2 = _sc_spmm(y2, src_pad, dst_pad, zeros64, 64)
    out = _tc_tail(z2[0], z2[1], y2, degA, degB, b2.reshape(1, -1))
    return out[:N]
